# Initial kernel scaffold; baseline (speedup 1.0000x reference)
import jax
import jax.numpy as jnp
from jax.experimental import pallas as pl


def kernel(x, edge_index, batch, W1, b1, W2, b2, fc_W, fc_b):
    raise NotImplementedError("write your pallas kernel here")


# speedup vs baseline: 14.6161x; 14.6161x over previous
import functools

import jax
import jax.numpy as jnp
from jax import lax
from jax.experimental import pallas as pl
from jax.experimental.pallas import tpu as pltpu
from jax.experimental.pallas import tpu_sc as plsc

N = 10000
E = 320000
D = 128
G = 64
DOUT = 10

NC = 2
NS = 16
NW = NC * NS
EPW = E // NW
CH = 128
NCH = -(-EPW // CH)
EPAD = NCH * CH
NP = 10112
RPS = NP // NS

_F32 = jnp.float32

_mesh = plsc.VectorSubcoreMesh(core_axis_name="c", subcore_axis_name="s")


@functools.partial(
    pl.kernel,
    mesh=_mesh,
    out_type=jax.ShapeDtypeStruct((NC, NP, D), _F32),
    scratch_types=[
        pltpu.VMEM((2, CH), jnp.int32),
        pltpu.VMEM((2, CH), jnp.int32),
        pltpu.VMEM((CH, D), _F32),
        pltpu.VMEM_SHARED((NP, D), _F32),
        pltpu.SemaphoreType.DMA,
        pltpu.SemaphoreType.DMA,
    ],
)
def _deg_sc(sidx_hbm, ones_hbm, zeros_hbm, out_hbm, i0, i1, ones_v, acc,
            isem0, isem1):
    c = lax.axis_index("c")
    s = lax.axis_index("s")
    wid = c * NS + s
    r0 = s * RPS
    pltpu.sync_copy(zeros_hbm.at[pl.ds(r0, RPS)], acc.at[pl.ds(r0, RPS)])
    pltpu.sync_copy(ones_hbm, ones_v)
    plsc.subcore_barrier()

    def _idx(j, ibuf, sem):
        return pltpu.make_async_copy(sidx_hbm.at[wid, j], ibuf, sem)

    assert NCH % 2 == 1
    _idx(0, i0, isem0).start()

    @pl.loop(0, NCH - 1, step=2)
    def _(j):
        _idx(j, i0, isem0).wait()
        _idx(j + 1, i1, isem1).start()
        pltpu.sync_copy(ones_v, acc.at[i0.at[1]], add=True)
        _idx(j + 1, i1, isem1).wait()
        _idx(j + 2, i0, isem0).start()
        pltpu.sync_copy(ones_v, acc.at[i1.at[1]], add=True)

    _idx(NCH - 1, i0, isem0).wait()
    pltpu.sync_copy(ones_v, acc.at[i0.at[1]], add=True)

    plsc.subcore_barrier()
    pltpu.sync_copy(acc.at[pl.ds(r0, RPS)], out_hbm.at[c, pl.ds(r0, RPS)])


@functools.partial(
    pl.kernel,
    mesh=_mesh,
    out_type=jax.ShapeDtypeStruct((NC, NP, D), _F32),
    scratch_types=[
        pltpu.VMEM((2, CH), jnp.int32),
        pltpu.VMEM((2, CH), jnp.int32),
        pltpu.VMEM((CH, D), _F32),
        pltpu.VMEM((CH, D), _F32),
        pltpu.VMEM_SHARED((NP, D), _F32),
        pltpu.SemaphoreType.DMA,
        pltpu.SemaphoreType.DMA,
        pltpu.SemaphoreType.DMA,
        pltpu.SemaphoreType.DMA,
    ],
)
def _gs_sc(p_hbm, sidx_hbm, zeros_hbm, out_hbm,
           i0, i1, b0, b1, acc, gsem0, gsem1, isem0, isem1):
    c = lax.axis_index("c")
    s = lax.axis_index("s")
    wid = c * NS + s
    r0 = s * RPS
    pltpu.sync_copy(zeros_hbm.at[pl.ds(r0, RPS)], acc.at[pl.ds(r0, RPS)])
    plsc.subcore_barrier()

    def _idx(j, ibuf, sem):
        return pltpu.make_async_copy(sidx_hbm.at[wid, j], ibuf, sem)

    def _gather(ibuf, buf, sem):
        return pltpu.make_async_copy(p_hbm.at[ibuf.at[0]], buf, sem)

    assert NCH % 2 == 1
    _idx(0, i0, isem0).start()
    _idx(0, i0, isem0).wait()
    _gather(i0, b0, gsem0).start()
    _idx(1, i1, isem1).start()

    @pl.loop(0, NCH - 1, step=2)
    def _(j):
        _idx(j + 1, i1, isem1).wait()
        _gather(i1, b1, gsem1).start()
        _gather(i0, b0, gsem0).wait()
        pltpu.sync_copy(b0, acc.at[i0.at[1]], add=True)
        _idx(j + 2, i0, isem0).start()
        _gather(i1, b1, gsem1).wait()
        pltpu.sync_copy(b1, acc.at[i1.at[1]], add=True)
        _idx(j + 3, i1, isem1).start()
        _idx(j + 2, i0, isem0).wait()
        _gather(i0, b0, gsem0).start()

    _gather(i0, b0, gsem0).wait()
    pltpu.sync_copy(b0, acc.at[i0.at[1]], add=True)
    _idx(NCH, i1, isem1).wait()

    plsc.subcore_barrier()
    pltpu.sync_copy(acc.at[pl.ds(r0, RPS)], out_hbm.at[c, pl.ds(r0, RPS)])


def _tc_mm(x, w):
    def body(x_ref, w_ref, o_ref):
        o_ref[...] = jnp.dot(x_ref[...], w_ref[...],
                             preferred_element_type=_F32,
                             precision=lax.Precision.HIGHEST)

    return pl.pallas_call(
        body, out_shape=jax.ShapeDtypeStruct((x.shape[0], w.shape[1]), _F32),
    )(x, w)


def _tc_scale(cnt, h1):
    def body(cnt_ref, h_ref, dinv_ref, p_ref):
        deg = cnt_ref[0, 0:N, 0:1] + cnt_ref[1, 0:N, 0:1] + 1.0
        dinv = lax.rsqrt(deg)
        dinv_ref[...] = dinv
        p_ref[0:N, :] = h_ref[...] * dinv
        p_ref[N:NP, :] = jnp.zeros((NP - N, D), _F32)

    return pl.pallas_call(
        body,
        out_shape=[jax.ShapeDtypeStruct((N, 1), _F32),
                   jax.ShapeDtypeStruct((NP, D), _F32)],
    )(cnt, h1)


def _tc_mid(s1, p1, dinv, b, w):
    def body(s_ref, p_ref, dinv_ref, b_ref, w_ref, p2_ref):
        u = s_ref[0, 0:N, :] + s_ref[1, 0:N, :] + p_ref[0:N, :]
        h = jnp.maximum(u * dinv_ref[...] + b_ref[...], 0.0)
        h2 = jnp.dot(h, w_ref[...], preferred_element_type=_F32,
                     precision=lax.Precision.HIGHEST)
        p2_ref[0:N, :] = h2 * dinv_ref[...]
        p2_ref[N:NP, :] = jnp.zeros((NP - N, D), _F32)

    return pl.pallas_call(
        body, out_shape=jax.ShapeDtypeStruct((NP, D), _F32),
    )(s1, p1, dinv, b, w)


def _tc_final(s2, p2, dinv, b, batch2, fc_w, fc_b):
    def body(s_ref, p_ref, dinv_ref, b_ref, batch_ref, fcw_ref, fcb_ref,
             o_ref):
        u = s_ref[0, 0:N, :] + s_ref[1, 0:N, :] + p_ref[0:N, :]
        h = jnp.maximum(u * dinv_ref[...] + b_ref[...], 0.0)
        gids = lax.broadcasted_iota(jnp.int32, (N, G), 1)
        m = (batch_ref[...] == gids).astype(_F32)
        gsum = lax.dot_general(m, h, (((0,), (0,)), ((), ())),
                               preferred_element_type=_F32,
                               precision=lax.Precision.HIGHEST)
        counts = jnp.sum(m, axis=0)[:, None]
        mean = gsum / jnp.maximum(counts, 1.0)
        logits = jnp.dot(mean, fcw_ref[...], preferred_element_type=_F32,
                         precision=lax.Precision.HIGHEST) + fcb_ref[...]
        mx = jnp.max(logits, axis=1, keepdims=True)
        lse = jnp.log(jnp.sum(jnp.exp(logits - mx), axis=1,
                              keepdims=True)) + mx
        o_ref[...] = logits - lse

    return pl.pallas_call(
        body, out_shape=jax.ShapeDtypeStruct((G, DOUT), _F32),
    )(s2, p2, dinv, b, batch2, fc_w, fc_b)


def kernel(x, edge_index, batch, W1, b1, W2, b2, fc_W, fc_b):
    ei = edge_index.astype(jnp.int32)
    pad = jnp.full((NW, EPAD - EPW), N, jnp.int32)
    src3 = jnp.concatenate([ei[0].reshape(NW, EPW), pad], axis=1)
    src3 = src3.reshape(NW, NCH, CH)
    dst3 = jnp.concatenate([ei[1].reshape(NW, EPW), pad], axis=1)
    dst3 = dst3.reshape(NW, NCH, CH)
    sidx = jnp.stack([src3, dst3], axis=2)
    padc = jnp.full((NW, 1, 2, CH), N, jnp.int32)
    sidx = jnp.concatenate([sidx, padc], axis=1)

    onesD = jnp.ones((CH, D), _F32)
    zerosD = jnp.zeros((NP, D), _F32)
    batch2 = batch.astype(jnp.int32).reshape(N, 1)
    b1r = b1.reshape(1, D)
    b2r = b2.reshape(1, D)
    fcbr = fc_b.reshape(1, DOUT)

    cnt = _deg_sc(sidx, onesD, zerosD)
    h1 = _tc_mm(x, W1)
    dinv, p1 = _tc_scale(cnt, h1)
    s1 = _gs_sc(p1, sidx, zerosD)
    p2 = _tc_mid(s1, p1, dinv, b1r, W2)
    s2 = _gs_sc(p2, sidx, zerosD)
    return _tc_final(s2, p2, dinv, b2r, batch2, fc_W, fcbr)
